# baseline (speedup 1.0000x reference)
import functools

import jax
import jax.numpy as jnp
from jax import lax
from jax.experimental import pallas as pl
from jax.experimental.pallas import tpu as pltpu
from jax.experimental.pallas import tpu_sc as plsc

BSZ = 4
SEQ_LEN = 8192
EMBED_DIM = 768

_NC = 2
_NS = 16
_NW = _NC * _NS
_ROWS_PER_W = SEQ_LEN // _NW


def _sc_body(wpe_hbm, out_hbm, sem):
    wid = lax.axis_index("s") * _NC + lax.axis_index("c")
    base = wid * _ROWS_PER_W
    copies = [
        pltpu.make_async_copy(
            wpe_hbm.at[pl.ds(base, _ROWS_PER_W)],
            out_hbm.at[b, pl.ds(base, _ROWS_PER_W)],
            sem,
        )
        for b in range(BSZ)
    ]
    for c in copies:
        c.start()
    for c in copies:
        c.wait()


def kernel(tokens, wpe):
    del tokens
    run = functools.partial(
        pl.kernel,
        mesh=plsc.VectorSubcoreMesh(core_axis_name="c", subcore_axis_name="s"),
        out_type=jax.ShapeDtypeStruct((BSZ, SEQ_LEN, EMBED_DIM), jnp.float32),
        scratch_types=[pltpu.SemaphoreType.DMA],
    )(_sc_body)
    return run(wpe)


# speedup vs baseline: 51.6358x; 51.6358x over previous
import functools

import jax
import jax.numpy as jnp
from jax import lax
from jax.experimental import pallas as pl
from jax.experimental.pallas import tpu as pltpu
from jax.experimental.pallas import tpu_sc as plsc

BSZ = 4
SEQ_LEN = 8192
EMBED_DIM = 768

_NC = 2
_NS = 16
_NW = _NC * _NS
_ROWS_PER_W = SEQ_LEN // _NW


_CHUNK = 128
_NCHUNK = _ROWS_PER_W // _CHUNK


def _sc_body(wpe_hbm, out_hbm, stage, sem_in, sem_out):
    wid = lax.axis_index("s") * _NC + lax.axis_index("c")
    base = wid * _ROWS_PER_W
    for ci in range(_NCHUNK):
        off = base + ci * _CHUNK
        pltpu.sync_copy(wpe_hbm.at[pl.ds(off, _CHUNK)], stage)
        outs = [
            pltpu.make_async_copy(
                stage, out_hbm.at[b, pl.ds(off, _CHUNK)], sem_out
            )
            for b in range(BSZ)
        ]
        for c in outs:
            c.start()
        for c in outs:
            c.wait()


def kernel(tokens, wpe):
    del tokens
    run = functools.partial(
        pl.kernel,
        mesh=plsc.VectorSubcoreMesh(core_axis_name="c", subcore_axis_name="s"),
        out_type=jax.ShapeDtypeStruct((BSZ, SEQ_LEN, EMBED_DIM), jnp.float32),
        scratch_types=[
            pltpu.VMEM((_CHUNK, EMBED_DIM), jnp.float32),
            pltpu.SemaphoreType.DMA,
            pltpu.SemaphoreType.DMA,
        ],
    )(_sc_body)
    return run(wpe)
